# Initial kernel scaffold; baseline (speedup 1.0000x reference)
import jax
import jax.numpy as jnp
from jax.experimental import pallas as pl


def kernel(features, edge_index, W1, b1, W2, b2, Wo1, bo1, Wo2, bo2):
    raise NotImplementedError("write your pallas kernel here")


# speedup vs baseline: 5.1936x; 5.1936x over previous
import jax
import jax.numpy as jnp
from jax.experimental import pallas as pl
from jax.experimental.pallas import tpu as pltpu
from jax.experimental.pallas import tpu_sc as plsc

N = 10000
E = 320000
IN = 128
D = 64
WD = 128
NC = 2
NS = 16
NW = NC * NS
EPW = E // NW
K = 80
NB = EPW // K
CW = 10
RPS = N // CW
BN = 400


def _fill_ones(ref, rows, width):
    @pl.loop(0, rows)
    def _(r):
        for c in range(width // 16):
            ref[r, pl.ds(c * 16, 16)] = jnp.ones((16,), jnp.float32)


def _deg_body(src_hbm, dst_hbm, z_hbm, outa_hbm, outb_hbm,
              idxs_v, idxd_v, ones_v, acc_sh):
    c = jax.lax.axis_index("c")
    s = jax.lax.axis_index("s")
    w = c * NS + s
    pltpu.sync_copy(src_hbm.at[w], idxs_v)
    pltpu.sync_copy(dst_hbm.at[w], idxd_v)
    _fill_ones(ones_v, K, WD)

    def phase(idx_v, out_hbm):
        @pl.when(s < CW)
        def _():
            pltpu.sync_copy(z_hbm.at[pl.ds(s * RPS, RPS)],
                            acc_sh.at[pl.ds(s * RPS, RPS)])

        plsc.subcore_barrier()

        @pl.loop(0, NB)
        def _(j):
            pltpu.sync_copy(ones_v, acc_sh.at[idx_v.at[j]], add=True)

        plsc.subcore_barrier()

        @pl.when(s < CW)
        def _():
            pltpu.sync_copy(acc_sh.at[pl.ds(s * RPS, RPS)],
                            out_hbm.at[c, pl.ds(s * RPS, RPS)])

        plsc.subcore_barrier()

    phase(idxs_v, outa_hbm)
    phase(idxd_v, outb_hbm)


_DEG = pl.kernel(
    _deg_body,
    out_type=(jax.ShapeDtypeStruct((NC, N, WD), jnp.float32),
              jax.ShapeDtypeStruct((NC, N, WD), jnp.float32)),
    mesh=plsc.VectorSubcoreMesh(core_axis_name="c", subcore_axis_name="s",
                                num_cores=NC, num_subcores=NS),
    scratch_types=[
        pltpu.VMEM((NB, K), jnp.int32),
        pltpu.VMEM((NB, K), jnp.int32),
        pltpu.VMEM((K, WD), jnp.float32),
        pltpu.VMEM_SHARED((N, WD), jnp.float32),
    ],
)


def _seg_body(h_hbm, src_hbm, dst_hbm, z_hbm, out_hbm,
              idxs_v, idxd_v, rows_v, acc_sh):
    c = jax.lax.axis_index("c")
    s = jax.lax.axis_index("s")
    w = c * NS + s
    pltpu.sync_copy(src_hbm.at[w], idxs_v)
    pltpu.sync_copy(dst_hbm.at[w], idxd_v)

    @pl.when(s < CW)
    def _():
        pltpu.sync_copy(z_hbm.at[pl.ds(s * RPS, RPS)],
                        acc_sh.at[pl.ds(s * RPS, RPS)])

    plsc.subcore_barrier()

    @pl.loop(0, NB)
    def _(j):
        pltpu.sync_copy(h_hbm.at[idxs_v.at[j]], rows_v)
        pltpu.sync_copy(rows_v, acc_sh.at[idxd_v.at[j]], add=True)

    plsc.subcore_barrier()

    @pl.when(s < CW)
    def _():
        pltpu.sync_copy(acc_sh.at[pl.ds(s * RPS, RPS)],
                        out_hbm.at[c, pl.ds(s * RPS, RPS)])


_SEG = pl.kernel(
    _seg_body,
    out_type=jax.ShapeDtypeStruct((NC, N, WD), jnp.float32),
    mesh=plsc.VectorSubcoreMesh(core_axis_name="c", subcore_axis_name="s",
                                num_cores=NC, num_subcores=NS),
    scratch_types=[
        pltpu.VMEM((NB, K), jnp.int32),
        pltpu.VMEM((NB, K), jnp.int32),
        pltpu.VMEM((K, WD), jnp.float32),
        pltpu.VMEM_SHARED((N, WD), jnp.float32),
    ],
)


def _norm(dcol):
    return jnp.where(dcol > 0, jax.lax.rsqrt(jnp.maximum(dcol, 1.0)), 0.0)


def _b_body(dega_ref, x_ref, w1_ref, o_ref):
    ns = _norm(dega_ref[0][:, 0:1] + dega_ref[1][:, 0:1])
    o_ref[...] = jax.lax.dot_general(
        x_ref[...] * ns, w1_ref[...], (((1,), (0,)), ((), ())),
        preferred_element_type=jnp.float32)


def _d_body(aggp_ref, dega_ref, degb_ref, b1_ref, w2_ref, o_ref):
    ns = _norm(dega_ref[0][:, 0:1] + dega_ref[1][:, 0:1])
    nd = _norm(degb_ref[0][:, 0:1] + degb_ref[1][:, 0:1])
    agg = aggp_ref[0][:, :D] + aggp_ref[1][:, :D]
    x1 = jnp.maximum(agg * nd + b1_ref[...], 0.0)
    o_ref[...] = jax.lax.dot_general(
        x1 * ns, w2_ref[...], (((1,), (0,)), ((), ())),
        preferred_element_type=jnp.float32)


def _f_body(aggp_ref, degb_ref, b2_ref, wo1_ref, bo1_ref, wo2_ref, bo2_ref,
            o_ref):
    nd = _norm(degb_ref[0][:, 0:1] + degb_ref[1][:, 0:1])
    x2 = (aggp_ref[0][:, :D] + aggp_ref[1][:, :D]) * nd + b2_ref[...]
    t = jnp.maximum(jax.lax.dot_general(
        x2, wo1_ref[...], (((1,), (0,)), ((), ())),
        preferred_element_type=jnp.float32) + bo1_ref[...], 0.0)
    o_ref[...] = jax.lax.dot_general(
        t, wo2_ref[...], (((1,), (0,)), ((), ())),
        preferred_element_type=jnp.float32) + bo2_ref[...]


_GRID = (N // BN,)
_deg_spec = pl.BlockSpec((NC, BN, WD), lambda i: (0, i, 0))
_agg_spec = pl.BlockSpec((NC, BN, WD), lambda i: (0, i, 0))


def _stage_b(dega, x, w1):
    return pl.pallas_call(
        _b_body,
        grid=_GRID,
        in_specs=[_deg_spec,
                  pl.BlockSpec((BN, IN), lambda i: (i, 0)),
                  pl.BlockSpec((IN, WD), lambda i: (0, 0))],
        out_specs=pl.BlockSpec((BN, WD), lambda i: (i, 0)),
        out_shape=jax.ShapeDtypeStruct((N, WD), jnp.float32),
    )(dega, x, w1)


def _stage_d(aggp, dega, degb, b1, w2):
    return pl.pallas_call(
        _d_body,
        grid=_GRID,
        in_specs=[_agg_spec, _deg_spec, _deg_spec,
                  pl.BlockSpec((1, D), lambda i: (0, 0)),
                  pl.BlockSpec((D, WD), lambda i: (0, 0))],
        out_specs=pl.BlockSpec((BN, WD), lambda i: (i, 0)),
        out_shape=jax.ShapeDtypeStruct((N, WD), jnp.float32),
    )(aggp, dega, degb, b1, w2)


def _stage_f(aggp, degb, b2, wo1, bo1, wo2, bo2):
    return pl.pallas_call(
        _f_body,
        grid=_GRID,
        in_specs=[_agg_spec, _deg_spec,
                  pl.BlockSpec((1, D), lambda i: (0, 0)),
                  pl.BlockSpec((D, D), lambda i: (0, 0)),
                  pl.BlockSpec((1, D), lambda i: (0, 0)),
                  pl.BlockSpec((D, 1), lambda i: (0, 0)),
                  pl.BlockSpec((1, 1), lambda i: (0, 0))],
        out_specs=pl.BlockSpec((BN, 1), lambda i: (i, 0)),
        out_shape=jax.ShapeDtypeStruct((N, 1), jnp.float32),
    )(aggp, degb, b2, wo1, bo1, wo2, bo2)


def kernel(features, edge_index, W1, b1, W2, b2, Wo1, bo1, Wo2, bo2):
    x = features.reshape(N, IN)
    srcb = edge_index[0].reshape(NW, NB, K)
    dstb = edge_index[1].reshape(NW, NB, K)
    z = jnp.zeros((N, WD), jnp.float32)
    w1p = jnp.pad(W1, ((0, 0), (0, WD - D)))
    w2p = jnp.pad(W2, ((0, 0), (0, WD - D)))
    dega, degb = _DEG(srcb, dstb, z)
    h1 = _stage_b(dega, x, w1p)
    agg1 = _SEG(h1, srcb, dstb, z)
    h2 = _stage_d(agg1, dega, degb, b1.reshape(1, D), w2p)
    agg2 = _SEG(h2, srcb, dstb, z)
    return _stage_f(agg2, degb, b2.reshape(1, D), Wo1, bo1.reshape(1, D),
                    Wo2.reshape(D, 1), bo2.reshape(1, 1))


# speedup vs baseline: 6.1560x; 1.1853x over previous
import jax
import jax.numpy as jnp
from jax.experimental import pallas as pl
from jax.experimental.pallas import tpu as pltpu
from jax.experimental.pallas import tpu_sc as plsc

N = 10000
E = 320000
IN = 128
D = 64
WD = 128
NC = 2
NS = 16
NW = NC * NS
EPW = E // NW
K = 80
NB = EPW // K
CW = 10
RPS = N // CW
BN = 400


def _fill_ones(ref, rows, width):
    @pl.loop(0, rows)
    def _(r):
        for c in range(width // 16):
            ref[r, pl.ds(c * 16, 16)] = jnp.ones((16,), jnp.float32)


def _deg_body(src_hbm, dst_hbm, z_hbm, outa_hbm, outb_hbm,
              idxs_v, idxd_v, ones_v, acc_sh, sem1, sem2):
    c = jax.lax.axis_index("c")
    s = jax.lax.axis_index("s")
    w = c * NS + s
    pltpu.sync_copy(src_hbm.at[w], idxs_v)
    pltpu.sync_copy(dst_hbm.at[w], idxd_v)
    _fill_ones(ones_v, K, WD)

    def phase(idx_v, out_hbm):
        @pl.when(s < CW)
        def _():
            pltpu.sync_copy(z_hbm.at[pl.ds(s * RPS, RPS)],
                            acc_sh.at[pl.ds(s * RPS, RPS)])

        plsc.subcore_barrier()

        @pl.loop(0, NB - 1, step=2)
        def _(j):
            d1 = pltpu.async_copy(ones_v, acc_sh.at[idx_v.at[j]], sem1,
                                  add=True)
            d2 = pltpu.async_copy(ones_v, acc_sh.at[idx_v.at[j + 1]], sem2,
                                  add=True)
            d1.wait()
            d2.wait()

        pltpu.sync_copy(ones_v, acc_sh.at[idx_v.at[NB - 1]], add=True)
        plsc.subcore_barrier()

        @pl.when(s < CW)
        def _():
            pltpu.sync_copy(acc_sh.at[pl.ds(s * RPS, RPS)],
                            out_hbm.at[c, pl.ds(s * RPS, RPS)])

        plsc.subcore_barrier()

    phase(idxs_v, outa_hbm)
    phase(idxd_v, outb_hbm)


_DEG = pl.kernel(
    _deg_body,
    out_type=(jax.ShapeDtypeStruct((NC, N, WD), jnp.float32),
              jax.ShapeDtypeStruct((NC, N, WD), jnp.float32)),
    mesh=plsc.VectorSubcoreMesh(core_axis_name="c", subcore_axis_name="s",
                                num_cores=NC, num_subcores=NS),
    scratch_types=[
        pltpu.VMEM((NB, K), jnp.int32),
        pltpu.VMEM((NB, K), jnp.int32),
        pltpu.VMEM((K, WD), jnp.float32),
        pltpu.VMEM_SHARED((N, WD), jnp.float32),
        pltpu.SemaphoreType.DMA,
        pltpu.SemaphoreType.DMA,
    ],
)


def _seg_body(h_hbm, src_hbm, dst_hbm, z_hbm, out_hbm,
              idxs_v, idxd_v, rows_a, rows_b, acc_sh,
              gsem_a, gsem_b, ssem_a, ssem_b):
    c = jax.lax.axis_index("c")
    s = jax.lax.axis_index("s")
    w = c * NS + s
    pltpu.sync_copy(src_hbm.at[w], idxs_v)
    pltpu.sync_copy(dst_hbm.at[w], idxd_v)

    @pl.when(s < CW)
    def _():
        pltpu.sync_copy(z_hbm.at[pl.ds(s * RPS, RPS)],
                        acc_sh.at[pl.ds(s * RPS, RPS)])

    plsc.subcore_barrier()

    pltpu.sync_copy(h_hbm.at[idxs_v.at[pl.ds(0, K)]], rows_a)

    @pl.loop(0, NB - 1, step=2)
    def _(p):
        dg = pltpu.async_copy(h_hbm.at[idxs_v.at[pl.ds((p + 1) * K, K)]],
                              rows_b, gsem_b)
        ds = pltpu.async_copy(rows_a, acc_sh.at[idxd_v.at[p]], ssem_a,
                              add=True)
        dg.wait()
        ds.wait()
        dg2 = pltpu.async_copy(h_hbm.at[idxs_v.at[pl.ds((p + 2) * K, K)]],
                               rows_a, gsem_a)
        ds2 = pltpu.async_copy(rows_b, acc_sh.at[idxd_v.at[p + 1]], ssem_b,
                               add=True)
        dg2.wait()
        ds2.wait()

    pltpu.sync_copy(rows_a, acc_sh.at[idxd_v.at[NB - 1]], add=True)
    plsc.subcore_barrier()

    @pl.when(s < CW)
    def _():
        pltpu.sync_copy(acc_sh.at[pl.ds(s * RPS, RPS)],
                        out_hbm.at[c, pl.ds(s * RPS, RPS)])


_SEG = pl.kernel(
    _seg_body,
    out_type=jax.ShapeDtypeStruct((NC, N, WD), jnp.float32),
    mesh=plsc.VectorSubcoreMesh(core_axis_name="c", subcore_axis_name="s",
                                num_cores=NC, num_subcores=NS),
    scratch_types=[
        pltpu.VMEM((EPW,), jnp.int32),
        pltpu.VMEM((NB, K), jnp.int32),
        pltpu.VMEM((K, WD), jnp.float32),
        pltpu.VMEM((K, WD), jnp.float32),
        pltpu.VMEM_SHARED((N, WD), jnp.float32),
        pltpu.SemaphoreType.DMA,
        pltpu.SemaphoreType.DMA,
        pltpu.SemaphoreType.DMA,
        pltpu.SemaphoreType.DMA,
    ],
)


def _norm(dcol):
    return jnp.where(dcol > 0, jax.lax.rsqrt(jnp.maximum(dcol, 1.0)), 0.0)


def _b_body(dega_ref, x_ref, w1_ref, o_ref):
    ns = _norm(dega_ref[0][:, 0:1] + dega_ref[1][:, 0:1])
    o_ref[...] = jax.lax.dot_general(
        x_ref[...] * ns, w1_ref[...], (((1,), (0,)), ((), ())),
        preferred_element_type=jnp.float32)


def _d_body(aggp_ref, dega_ref, degb_ref, b1_ref, w2_ref, o_ref):
    ns = _norm(dega_ref[0][:, 0:1] + dega_ref[1][:, 0:1])
    nd = _norm(degb_ref[0][:, 0:1] + degb_ref[1][:, 0:1])
    agg = aggp_ref[0][:, :D] + aggp_ref[1][:, :D]
    x1 = jnp.maximum(agg * nd + b1_ref[...], 0.0)
    o_ref[...] = jax.lax.dot_general(
        x1 * ns, w2_ref[...], (((1,), (0,)), ((), ())),
        preferred_element_type=jnp.float32)


def _f_body(aggp_ref, degb_ref, b2_ref, wo1_ref, bo1_ref, wo2_ref, bo2_ref,
            o_ref):
    nd = _norm(degb_ref[0][:, 0:1] + degb_ref[1][:, 0:1])
    x2 = (aggp_ref[0][:, :D] + aggp_ref[1][:, :D]) * nd + b2_ref[...]
    t = jnp.maximum(jax.lax.dot_general(
        x2, wo1_ref[...], (((1,), (0,)), ((), ())),
        preferred_element_type=jnp.float32) + bo1_ref[...], 0.0)
    o_ref[...] = jax.lax.dot_general(
        t, wo2_ref[...], (((1,), (0,)), ((), ())),
        preferred_element_type=jnp.float32) + bo2_ref[...]


_GRID = (N // BN,)
_deg_spec = pl.BlockSpec((NC, BN, WD), lambda i: (0, i, 0))
_agg_spec = pl.BlockSpec((NC, BN, WD), lambda i: (0, i, 0))


def _stage_b(dega, x, w1):
    return pl.pallas_call(
        _b_body,
        grid=_GRID,
        in_specs=[_deg_spec,
                  pl.BlockSpec((BN, IN), lambda i: (i, 0)),
                  pl.BlockSpec((IN, WD), lambda i: (0, 0))],
        out_specs=pl.BlockSpec((BN, WD), lambda i: (i, 0)),
        out_shape=jax.ShapeDtypeStruct((N, WD), jnp.float32),
    )(dega, x, w1)


def _stage_d(aggp, dega, degb, b1, w2):
    return pl.pallas_call(
        _d_body,
        grid=_GRID,
        in_specs=[_agg_spec, _deg_spec, _deg_spec,
                  pl.BlockSpec((1, D), lambda i: (0, 0)),
                  pl.BlockSpec((D, WD), lambda i: (0, 0))],
        out_specs=pl.BlockSpec((BN, WD), lambda i: (i, 0)),
        out_shape=jax.ShapeDtypeStruct((N, WD), jnp.float32),
    )(aggp, dega, degb, b1, w2)


def _stage_f(aggp, degb, b2, wo1, bo1, wo2, bo2):
    return pl.pallas_call(
        _f_body,
        grid=_GRID,
        in_specs=[_agg_spec, _deg_spec,
                  pl.BlockSpec((1, D), lambda i: (0, 0)),
                  pl.BlockSpec((D, D), lambda i: (0, 0)),
                  pl.BlockSpec((1, D), lambda i: (0, 0)),
                  pl.BlockSpec((D, 1), lambda i: (0, 0)),
                  pl.BlockSpec((1, 1), lambda i: (0, 0))],
        out_specs=pl.BlockSpec((BN, 1), lambda i: (i, 0)),
        out_shape=jax.ShapeDtypeStruct((N, 1), jnp.float32),
    )(aggp, degb, b2, wo1, bo1, wo2, bo2)


def kernel(features, edge_index, W1, b1, W2, b2, Wo1, bo1, Wo2, bo2):
    x = features.reshape(N, IN)
    srcb = edge_index[0].reshape(NW, NB, K)
    srcf = edge_index[0].reshape(NW, EPW)
    dstb = edge_index[1].reshape(NW, NB, K)
    z = jnp.zeros((N, WD), jnp.float32)
    w1p = jnp.pad(W1, ((0, 0), (0, WD - D)))
    w2p = jnp.pad(W2, ((0, 0), (0, WD - D)))
    dega, degb = _DEG(srcb, dstb, z)
    h1 = _stage_b(dega, x, w1p)
    agg1 = _SEG(h1, srcf, dstb, z)
    h2 = _stage_d(agg1, dega, degb, b1.reshape(1, D), w2p)
    agg2 = _SEG(h2, srcf, dstb, z)
    return _stage_f(agg2, degb, b2.reshape(1, D), Wo1, bo1.reshape(1, D),
                    Wo2.reshape(D, 1), bo2.reshape(1, 1))


# speedup vs baseline: 7.1031x; 1.1539x over previous
import jax
import jax.numpy as jnp
from jax.experimental import pallas as pl
from jax.experimental.pallas import tpu as pltpu
from jax.experimental.pallas import tpu_sc as plsc

N = 10000
E = 320000
IN = 128
D = 64
WD = 128
NC = 2
NS = 16
NW = NC * NS
EPW = E // NW
K = 80
NB = EPW // K
CW = 10
RPS = N // CW
BN = 400


def _fill_ones(ref, rows, width):
    @pl.loop(0, rows)
    def _(r):
        for c in range(width // 16):
            ref[r, pl.ds(c * 16, 16)] = jnp.ones((16,), jnp.float32)


def _deg_body(src_hbm, z_hbm, out_hbm, idx_v, ones_v, acc_sh, sem1, sem2):
    c = jax.lax.axis_index("c")
    s = jax.lax.axis_index("s")
    w = c * NS + s
    pltpu.sync_copy(src_hbm.at[w], idx_v)
    _fill_ones(ones_v, K, WD)

    @pl.when(s < CW)
    def _():
        pltpu.sync_copy(z_hbm.at[pl.ds(s * RPS, RPS)],
                        acc_sh.at[pl.ds(s * RPS, RPS)])

    plsc.subcore_barrier()

    @pl.loop(0, NB - 1, step=2)
    def _(j):
        d1 = pltpu.async_copy(ones_v, acc_sh.at[idx_v.at[j]], sem1, add=True)
        d2 = pltpu.async_copy(ones_v, acc_sh.at[idx_v.at[j + 1]], sem2,
                              add=True)
        d1.wait()
        d2.wait()

    pltpu.sync_copy(ones_v, acc_sh.at[idx_v.at[NB - 1]], add=True)
    plsc.subcore_barrier()

    @pl.when(s < CW)
    def _():
        pltpu.sync_copy(acc_sh.at[pl.ds(s * RPS, RPS)],
                        out_hbm.at[c, pl.ds(s * RPS, RPS)])


_DEG = pl.kernel(
    _deg_body,
    out_type=jax.ShapeDtypeStruct((NC, N, WD), jnp.float32),
    mesh=plsc.VectorSubcoreMesh(core_axis_name="c", subcore_axis_name="s",
                                num_cores=NC, num_subcores=NS),
    scratch_types=[
        pltpu.VMEM((NB, K), jnp.int32),
        pltpu.VMEM((K, WD), jnp.float32),
        pltpu.VMEM_SHARED((N, WD), jnp.float32),
        pltpu.SemaphoreType.DMA,
        pltpu.SemaphoreType.DMA,
    ],
)


def _seg_body(h_hbm, src_hbm, dst_hbm, z_hbm, out_hbm,
              idxs_v, idxd_v, rows_a, rows_b, acc_sh,
              gsem_a, gsem_b, ssem_a, ssem_b):
    c = jax.lax.axis_index("c")
    s = jax.lax.axis_index("s")
    w = c * NS + s
    pltpu.sync_copy(src_hbm.at[w], idxs_v)
    pltpu.sync_copy(dst_hbm.at[w], idxd_v)

    @pl.when(s < CW)
    def _():
        pltpu.sync_copy(z_hbm.at[pl.ds(s * RPS, RPS)],
                        acc_sh.at[pl.ds(s * RPS, RPS)])

    plsc.subcore_barrier()

    pltpu.sync_copy(h_hbm.at[idxs_v.at[pl.ds(0, K)]], rows_a)

    @pl.loop(0, NB - 1, step=2)
    def _(p):
        dg = pltpu.async_copy(h_hbm.at[idxs_v.at[pl.ds((p + 1) * K, K)]],
                              rows_b, gsem_b)
        ds = pltpu.async_copy(rows_a, acc_sh.at[idxd_v.at[p]], ssem_a,
                              add=True)
        dg.wait()
        ds.wait()
        dg2 = pltpu.async_copy(h_hbm.at[idxs_v.at[pl.ds((p + 2) * K, K)]],
                               rows_a, gsem_a)
        ds2 = pltpu.async_copy(rows_b, acc_sh.at[idxd_v.at[p + 1]], ssem_b,
                               add=True)
        dg2.wait()
        ds2.wait()

    pltpu.sync_copy(rows_a, acc_sh.at[idxd_v.at[NB - 1]], add=True)
    plsc.subcore_barrier()

    @pl.when(s < CW)
    def _():
        pltpu.sync_copy(acc_sh.at[pl.ds(s * RPS, RPS)],
                        out_hbm.at[c, pl.ds(s * RPS, RPS)])


_SEG = pl.kernel(
    _seg_body,
    out_type=jax.ShapeDtypeStruct((NC, N, WD), jnp.float32),
    mesh=plsc.VectorSubcoreMesh(core_axis_name="c", subcore_axis_name="s",
                                num_cores=NC, num_subcores=NS),
    scratch_types=[
        pltpu.VMEM((EPW,), jnp.int32),
        pltpu.VMEM((NB, K), jnp.int32),
        pltpu.VMEM((K, WD), jnp.float32),
        pltpu.VMEM((K, WD), jnp.float32),
        pltpu.VMEM_SHARED((N, WD), jnp.float32),
        pltpu.SemaphoreType.DMA,
        pltpu.SemaphoreType.DMA,
        pltpu.SemaphoreType.DMA,
        pltpu.SemaphoreType.DMA,
    ],
)


def _norm(dcol):
    return jnp.where(dcol > 0, jax.lax.rsqrt(jnp.maximum(dcol, 1.0)), 0.0)


def _col64(shape):
    lane = jax.lax.broadcasted_iota(jnp.int32, shape, 1)
    return jnp.where(lane == D, 1.0, 0.0).astype(jnp.float32)


def _b_body(dega_ref, x_ref, w1_ref, o_ref):
    ns = _norm(dega_ref[0][:, 0:1] + dega_ref[1][:, 0:1])
    o_ref[...] = jax.lax.dot_general(
        x_ref[...] * ns, w1_ref[...], (((1,), (0,)), ((), ())),
        preferred_element_type=jnp.float32) + _col64((BN, WD))


def _d_body(aggp_ref, dega_ref, b1_ref, w2_ref, o_ref):
    ns = _norm(dega_ref[0][:, 0:1] + dega_ref[1][:, 0:1])
    nd = _norm(aggp_ref[0][:, D:D + 1] + aggp_ref[1][:, D:D + 1])
    agg = aggp_ref[0][:, :D] + aggp_ref[1][:, :D]
    x1 = jnp.maximum(agg * nd + b1_ref[...], 0.0)
    o_ref[...] = jax.lax.dot_general(
        x1 * ns, w2_ref[...], (((1,), (0,)), ((), ())),
        preferred_element_type=jnp.float32) + _col64((BN, WD))


def _f_body(aggp_ref, b2_ref, wo1_ref, bo1_ref, wo2_ref, bo2_ref, o_ref):
    nd = _norm(aggp_ref[0][:, D:D + 1] + aggp_ref[1][:, D:D + 1])
    x2 = (aggp_ref[0][:, :D] + aggp_ref[1][:, :D]) * nd + b2_ref[...]
    t = jnp.maximum(jax.lax.dot_general(
        x2, wo1_ref[...], (((1,), (0,)), ((), ())),
        preferred_element_type=jnp.float32) + bo1_ref[...], 0.0)
    o_ref[...] = jax.lax.dot_general(
        t, wo2_ref[...], (((1,), (0,)), ((), ())),
        preferred_element_type=jnp.float32) + bo2_ref[...]


_GRID = (N // BN,)
_deg_spec = pl.BlockSpec((NC, BN, WD), lambda i: (0, i, 0))
_agg_spec = pl.BlockSpec((NC, BN, WD), lambda i: (0, i, 0))


def _stage_b(dega, x, w1):
    return pl.pallas_call(
        _b_body,
        grid=_GRID,
        in_specs=[_deg_spec,
                  pl.BlockSpec((BN, IN), lambda i: (i, 0)),
                  pl.BlockSpec((IN, WD), lambda i: (0, 0))],
        out_specs=pl.BlockSpec((BN, WD), lambda i: (i, 0)),
        out_shape=jax.ShapeDtypeStruct((N, WD), jnp.float32),
    )(dega, x, w1)


def _stage_d(aggp, dega, b1, w2):
    return pl.pallas_call(
        _d_body,
        grid=_GRID,
        in_specs=[_agg_spec, _deg_spec,
                  pl.BlockSpec((1, D), lambda i: (0, 0)),
                  pl.BlockSpec((D, WD), lambda i: (0, 0))],
        out_specs=pl.BlockSpec((BN, WD), lambda i: (i, 0)),
        out_shape=jax.ShapeDtypeStruct((N, WD), jnp.float32),
    )(aggp, dega, b1, w2)


def _stage_f(aggp, b2, wo1, bo1, wo2, bo2):
    return pl.pallas_call(
        _f_body,
        grid=_GRID,
        in_specs=[_agg_spec,
                  pl.BlockSpec((1, D), lambda i: (0, 0)),
                  pl.BlockSpec((D, D), lambda i: (0, 0)),
                  pl.BlockSpec((1, D), lambda i: (0, 0)),
                  pl.BlockSpec((D, 1), lambda i: (0, 0)),
                  pl.BlockSpec((1, 1), lambda i: (0, 0))],
        out_specs=pl.BlockSpec((BN, 1), lambda i: (i, 0)),
        out_shape=jax.ShapeDtypeStruct((N, 1), jnp.float32),
    )(aggp, b2, wo1, bo1, wo2, bo2)


def kernel(features, edge_index, W1, b1, W2, b2, Wo1, bo1, Wo2, bo2):
    x = features.reshape(N, IN)
    srcb = edge_index[0].reshape(NW, NB, K)
    srcf = edge_index[0].reshape(NW, EPW)
    dstb = edge_index[1].reshape(NW, NB, K)
    z = jnp.zeros((N, WD), jnp.float32)
    w1p = jnp.pad(W1, ((0, 0), (0, WD - D)))
    w2p = jnp.pad(W2, ((0, 0), (0, WD - D)))
    dega = _DEG(srcb, z)
    h1 = _stage_b(dega, x, w1p)
    agg1 = _SEG(h1, srcf, dstb, z)
    h2 = _stage_d(agg1, dega, b1.reshape(1, D), w2p)
    agg2 = _SEG(h2, srcf, dstb, z)
    return _stage_f(agg2, b2.reshape(1, D), Wo1, bo1.reshape(1, D),
                    Wo2.reshape(D, 1), bo2.reshape(1, 1))


# speedup vs baseline: 7.3391x; 1.0332x over previous
import jax
import jax.numpy as jnp
from jax.experimental import pallas as pl
from jax.experimental.pallas import tpu as pltpu
from jax.experimental.pallas import tpu_sc as plsc

N = 10000
E = 320000
IN = 128
D = 64
WD = 128
NC = 2
NS = 16
NW = NC * NS
K = 104
NB = 97
EPW = K * NB
EP = NW * EPW
NG = 8
NH = N + NG
CW = 10
RPS = N // CW
BN = 400


def _fill_ones(ref, rows, width):
    @pl.loop(0, rows)
    def _(r):
        for c in range(width // 16):
            ref[r, pl.ds(c * 16, 16)] = jnp.ones((16,), jnp.float32)


def _deg_body(src_hbm, z_hbm, out_hbm, idx_v, ones_v, acc_sh, sem1, sem2):
    c = jax.lax.axis_index("c")
    s = jax.lax.axis_index("s")
    w = c * NS + s
    pltpu.sync_copy(src_hbm.at[w], idx_v)
    _fill_ones(ones_v, K, WD)

    @pl.when(s < CW)
    def _():
        pltpu.sync_copy(z_hbm.at[pl.ds(s * RPS, RPS)],
                        acc_sh.at[pl.ds(s * RPS, RPS)])

    plsc.subcore_barrier()

    @pl.loop(0, NB - 1, step=2)
    def _(j):
        d1 = pltpu.async_copy(ones_v, acc_sh.at[idx_v.at[j]], sem1, add=True)
        d2 = pltpu.async_copy(ones_v, acc_sh.at[idx_v.at[j + 1]], sem2,
                              add=True)
        d1.wait()
        d2.wait()

    pltpu.sync_copy(ones_v, acc_sh.at[idx_v.at[NB - 1]], add=True)
    plsc.subcore_barrier()

    @pl.when(s < CW)
    def _():
        pltpu.sync_copy(acc_sh.at[pl.ds(s * RPS, RPS)],
                        out_hbm.at[c, pl.ds(s * RPS, RPS)])


_DEG = pl.kernel(
    _deg_body,
    out_type=jax.ShapeDtypeStruct((NC, N, WD), jnp.float32),
    mesh=plsc.VectorSubcoreMesh(core_axis_name="c", subcore_axis_name="s",
                                num_cores=NC, num_subcores=NS),
    scratch_types=[
        pltpu.VMEM((NB, K), jnp.int32),
        pltpu.VMEM((K, WD), jnp.float32),
        pltpu.VMEM_SHARED((NH, WD), jnp.float32),
        pltpu.SemaphoreType.DMA,
        pltpu.SemaphoreType.DMA,
    ],
)


def _seg_body(h_hbm, src_hbm, dst_hbm, z_hbm, out_hbm,
              idxs_v, idxd_v, rows_a, rows_b, acc_sh,
              gsem_a, gsem_b, ssem_a, ssem_b):
    c = jax.lax.axis_index("c")
    s = jax.lax.axis_index("s")
    w = c * NS + s
    pltpu.sync_copy(src_hbm.at[w], idxs_v)
    pltpu.sync_copy(dst_hbm.at[w], idxd_v)

    @pl.when(s < CW)
    def _():
        pltpu.sync_copy(z_hbm.at[pl.ds(s * RPS, RPS)],
                        acc_sh.at[pl.ds(s * RPS, RPS)])

    plsc.subcore_barrier()

    pltpu.sync_copy(h_hbm.at[idxs_v.at[pl.ds(0, K)]], rows_a)

    @pl.loop(0, NB - 1, step=2)
    def _(p):
        dg = pltpu.async_copy(h_hbm.at[idxs_v.at[pl.ds((p + 1) * K, K)]],
                              rows_b, gsem_b)
        ds = pltpu.async_copy(rows_a, acc_sh.at[idxd_v.at[p]], ssem_a,
                              add=True)
        dg.wait()
        ds.wait()
        dg2 = pltpu.async_copy(h_hbm.at[idxs_v.at[pl.ds((p + 2) * K, K)]],
                               rows_a, gsem_a)
        ds2 = pltpu.async_copy(rows_b, acc_sh.at[idxd_v.at[p + 1]], ssem_b,
                               add=True)
        dg2.wait()
        ds2.wait()

    pltpu.sync_copy(rows_a, acc_sh.at[idxd_v.at[NB - 1]], add=True)
    plsc.subcore_barrier()

    @pl.when(s < CW)
    def _():
        pltpu.sync_copy(acc_sh.at[pl.ds(s * RPS, RPS)],
                        out_hbm.at[c, pl.ds(s * RPS, RPS)])


_SEG = pl.kernel(
    _seg_body,
    out_type=jax.ShapeDtypeStruct((NC, N, WD), jnp.float32),
    mesh=plsc.VectorSubcoreMesh(core_axis_name="c", subcore_axis_name="s",
                                num_cores=NC, num_subcores=NS),
    scratch_types=[
        pltpu.VMEM((EPW,), jnp.int32),
        pltpu.VMEM((NB, K), jnp.int32),
        pltpu.VMEM((K, WD), jnp.float32),
        pltpu.VMEM((K, WD), jnp.float32),
        pltpu.VMEM_SHARED((NH, WD), jnp.float32),
        pltpu.SemaphoreType.DMA,
        pltpu.SemaphoreType.DMA,
        pltpu.SemaphoreType.DMA,
        pltpu.SemaphoreType.DMA,
    ],
)


def _norm(dcol):
    return jnp.where(dcol > 0, jax.lax.rsqrt(jnp.maximum(dcol, 1.0)), 0.0)


def _col64(shape):
    lane = jax.lax.broadcasted_iota(jnp.int32, shape, 1)
    return jnp.where(lane == D, 1.0, 0.0).astype(jnp.float32)


def _b_body(dega_ref, x_ref, w1_ref, o_ref):
    ns = _norm(dega_ref[0][:, 0:1] + dega_ref[1][:, 0:1])
    o_ref[...] = jax.lax.dot_general(
        x_ref[...] * ns, w1_ref[...], (((1,), (0,)), ((), ())),
        preferred_element_type=jnp.float32) + _col64((BN, WD))


def _d_body(aggp_ref, dega_ref, b1_ref, w2_ref, o_ref):
    ns = _norm(dega_ref[0][:, 0:1] + dega_ref[1][:, 0:1])
    nd = _norm(aggp_ref[0][:, D:D + 1] + aggp_ref[1][:, D:D + 1])
    agg = aggp_ref[0][:, :D] + aggp_ref[1][:, :D]
    x1 = jnp.maximum(agg * nd + b1_ref[...], 0.0)
    o_ref[...] = jax.lax.dot_general(
        x1 * ns, w2_ref[...], (((1,), (0,)), ((), ())),
        preferred_element_type=jnp.float32) + _col64((BN, WD))


def _f_body(aggp_ref, b2_ref, wo1_ref, bo1_ref, wo2_ref, bo2_ref, o_ref):
    nd = _norm(aggp_ref[0][:, D:D + 1] + aggp_ref[1][:, D:D + 1])
    x2 = (aggp_ref[0][:, :D] + aggp_ref[1][:, :D]) * nd + b2_ref[...]
    t = jnp.maximum(jax.lax.dot_general(
        x2, wo1_ref[...], (((1,), (0,)), ((), ())),
        preferred_element_type=jnp.float32) + bo1_ref[...], 0.0)
    o_ref[...] = jax.lax.dot_general(
        t, wo2_ref[...], (((1,), (0,)), ((), ())),
        preferred_element_type=jnp.float32) + bo2_ref[...]


_GRID = (N // BN,)
_deg_spec = pl.BlockSpec((NC, BN, WD), lambda i: (0, i, 0))
_agg_spec = pl.BlockSpec((NC, BN, WD), lambda i: (0, i, 0))


def _stage_b(dega, x, w1):
    return pl.pallas_call(
        _b_body,
        grid=_GRID,
        in_specs=[_deg_spec,
                  pl.BlockSpec((BN, IN), lambda i: (i, 0)),
                  pl.BlockSpec((IN, WD), lambda i: (0, 0))],
        out_specs=pl.BlockSpec((BN, WD), lambda i: (i, 0)),
        out_shape=jax.ShapeDtypeStruct((NH, WD), jnp.float32),
    )(dega, x, w1)


def _stage_d(aggp, dega, b1, w2):
    return pl.pallas_call(
        _d_body,
        grid=_GRID,
        in_specs=[_agg_spec, _deg_spec,
                  pl.BlockSpec((1, D), lambda i: (0, 0)),
                  pl.BlockSpec((D, WD), lambda i: (0, 0))],
        out_specs=pl.BlockSpec((BN, WD), lambda i: (i, 0)),
        out_shape=jax.ShapeDtypeStruct((NH, WD), jnp.float32),
    )(aggp, dega, b1, w2)


def _stage_f(aggp, b2, wo1, bo1, wo2, bo2):
    return pl.pallas_call(
        _f_body,
        grid=_GRID,
        in_specs=[_agg_spec,
                  pl.BlockSpec((1, D), lambda i: (0, 0)),
                  pl.BlockSpec((D, D), lambda i: (0, 0)),
                  pl.BlockSpec((1, D), lambda i: (0, 0)),
                  pl.BlockSpec((D, 1), lambda i: (0, 0)),
                  pl.BlockSpec((1, 1), lambda i: (0, 0))],
        out_specs=pl.BlockSpec((BN, 1), lambda i: (i, 0)),
        out_shape=jax.ShapeDtypeStruct((N, 1), jnp.float32),
    )(aggp, b2, wo1, bo1, wo2, bo2)


def kernel(features, edge_index, W1, b1, W2, b2, Wo1, bo1, Wo2, bo2):
    x = features.reshape(N, IN)
    gidx = N + (jnp.arange(EP - E, dtype=jnp.int32) % NG)
    src = jnp.concatenate([edge_index[0], gidx])
    dst = jnp.concatenate([edge_index[1], gidx])
    srcb = src.reshape(NW, NB, K)
    srcf = src.reshape(NW, EPW)
    dstb = dst.reshape(NW, NB, K)
    z = jnp.zeros((N, WD), jnp.float32)
    w1p = jnp.pad(W1, ((0, 0), (0, WD - D)))
    w2p = jnp.pad(W2, ((0, 0), (0, WD - D)))
    dega = _DEG(srcb, z)
    h1 = _stage_b(dega, x, w1p)
    agg1 = _SEG(h1, srcf, dstb, z)
    h2 = _stage_d(agg1, dega, b1.reshape(1, D), w2p)
    agg2 = _SEG(h2, srcf, dstb, z)
    return _stage_f(agg2, b2.reshape(1, D), Wo1, bo1.reshape(1, D),
                    Wo2.reshape(D, 1), bo2.reshape(1, 1))
